# baseline (speedup 1.0000x reference)
import functools

import jax
import jax.numpy as jnp
from jax import lax
from jax.experimental import pallas as pl
from jax.experimental.pallas import tpu as pltpu
from jax.experimental.pallas import tpu_sc as plsc

GEO = [1, 2, 3, 4, 5, 6, 8, 10, 22, 23, 29, 31, 32, 33, 43]
POS = [9, 16, 17, 20, 27, 30, 36, 42, 48, 49, 50]
SEM = [7, 11, 12, 13, 14, 15, 18, 19, 21, 24, 25, 26, 28, 34, 35, 37, 38,
       39, 40, 41, 44, 45, 46, 47]

NUM_OBJ = 5000
OBJ_PAD = 5120
NUM_REL = 20000
NUM_CLS = 151
N_CAT = 3 * NUM_REL
CAT_PAD = 61440
N_SORT = 65536
REL_PAD = 20480
NW = 32
REL_W = REL_PAD // NW


def _obj_body(logits_ref, score_ref, pred_ref):
    x = logits_ref[...]
    cc = x.shape[1]
    m = jnp.max(x, axis=0, keepdims=True)
    e = jnp.exp(x - m)
    t = jnp.concatenate([e, jnp.zeros((1, cc), jnp.float32)], axis=0)
    acc = t[0:8]
    for g in range(1, 19):
        acc = acc + t[8 * g:8 * g + 8]
    acc = acc[0:4] + acc[4:8]
    acc = acc[0:2] + acc[2:4]
    s = acc[0:1] + acc[1:2]
    p = e / s
    rowi = lax.broadcasted_iota(jnp.int32, x.shape, 0)
    p = jnp.where(rowi == 0, -1.0, p)
    mx = jnp.max(p, axis=0, keepdims=True)
    am = jnp.min(jnp.where(p == mx, rowi, NUM_CLS + 1), axis=0, keepdims=True)
    score_ref[...] = jnp.broadcast_to(mx, (8, cc))
    pred_ref[...] = jnp.broadcast_to(am, (8, cc))


def _obj_call(refine_logits_t):
    return pl.pallas_call(
        _obj_body,
        out_shape=[jax.ShapeDtypeStruct((8, NUM_OBJ), jnp.float32),
                   jax.ShapeDtypeStruct((8, NUM_OBJ), jnp.int32)],
    )(refine_logits_t)


def _branch_t(e, labels):
    mx = jnp.max(e, axis=0, keepdims=True)
    rowi = lax.broadcasted_iota(jnp.int32, e.shape, 0)
    am = jnp.min(jnp.where(e == mx, rowi, 1000), axis=0, keepdims=True)
    cls = jnp.full_like(am, labels[0])
    for k in range(1, len(labels)):
        cls = jnp.where(am == k, labels[k], cls)
    return mx, cls


def _rel_body(r1_ref, r2_ref, r3_ref, s_ref, c_ref):
    e1 = jnp.exp(r1_ref[...])
    e2 = jnp.exp(r2_ref[...])
    e3 = jnp.exp(r3_ref[...])
    s1, c1 = _branch_t(e1, GEO)
    s2, c2 = _branch_t(e2, POS)
    s3, c3 = _branch_t(e3, SEM)
    pc = c1 + c2 * 64 + c3 * 4096
    s_ref[...] = jnp.concatenate([s1, s2, s3, s3, s3, s3, s3, s3], axis=0)
    c_ref[...] = jnp.concatenate([pc, pc, pc, pc, pc, pc, pc, pc], axis=0)


def _rel_call(r1t, r2t, r3t):
    return pl.pallas_call(
        _rel_body,
        out_shape=[jax.ShapeDtypeStruct((8, NUM_REL), jnp.float32),
                   jax.ShapeDtypeStruct((8, NUM_REL), jnp.int32)],
    )(r1t, r2t, r3t)


def _rcat_body(r1_ref, r2_ref, r3_ref, rcat_ref):
    e1 = jnp.exp(r1_ref[...])
    e2 = jnp.exp(r2_ref[...])
    e3 = jnp.exp(r3_ref[...])
    pad = jnp.zeros((e1.shape[0], 78), jnp.float32)
    rcat_ref[...] = jnp.concatenate([e1, e2, e3, pad], axis=1)


def _rcat_call(r1, r2, r3):
    c = 2000
    return pl.pallas_call(
        _rcat_body,
        grid=(NUM_REL // c,),
        in_specs=[pl.BlockSpec((c, 15), lambda i: (i, 0)),
                  pl.BlockSpec((c, 11), lambda i: (i, 0)),
                  pl.BlockSpec((c, 24), lambda i: (i, 0))],
        out_specs=[pl.BlockSpec((c, 128), lambda i: (i, 0))],
        out_shape=[jax.ShapeDtypeStruct((NUM_REL, 128), jnp.float32)],
    )(r1, r2, r3)


def _keys_body(obj_hbm, idx0_hbm, idx1_hbm, s1_hbm, s2_hbm, s3_hbm,
               k1_hbm, k2_hbm, k3_hbm, pp_hbm,
               table_v, idx0_v, idx1_v, s1_v, s2_v, s3_v, k1_v, k2_v, k3_v,
               pp_v):
    wid = lax.axis_index("s") * 2 + lax.axis_index("c")
    base = wid * REL_W
    pltpu.sync_copy(obj_hbm, table_v)
    pltpu.sync_copy(idx0_hbm.at[pl.ds(base, REL_W)], idx0_v)
    pltpu.sync_copy(idx1_hbm.at[pl.ds(base, REL_W)], idx1_v)
    pltpu.sync_copy(s1_hbm.at[pl.ds(base, REL_W)], s1_v)
    pltpu.sync_copy(s2_hbm.at[pl.ds(base, REL_W)], s2_v)
    pltpu.sync_copy(s3_hbm.at[pl.ds(base, REL_W)], s3_v)
    for v in range(REL_W // 16):
        sl = pl.ds(v * 16, 16)
        i0 = idx0_v[sl]
        i1 = idx1_v[sl]
        g0 = plsc.load_gather(table_v, [i0])
        g1 = plsc.load_gather(table_v, [i1])
        k1_v[sl] = (s1_v[sl] * g0) * g1
        k2_v[sl] = (s2_v[sl] * g0) * g1
        k3_v[sl] = (s3_v[sl] * g0) * g1
        pp_v[sl] = i0 * 8192 + i1
    pltpu.sync_copy(k1_v, k1_hbm.at[pl.ds(base, REL_W)])
    pltpu.sync_copy(k2_v, k2_hbm.at[pl.ds(base, REL_W)])
    pltpu.sync_copy(k3_v, k3_hbm.at[pl.ds(base, REL_W)])
    pltpu.sync_copy(pp_v, pp_hbm.at[pl.ds(base, REL_W)])


def _keys_call(obj_scores, idx0, idx1, s1, s2, s3):
    f32 = jnp.float32
    i32 = jnp.int32
    kfn = pl.kernel(
        _keys_body,
        out_type=[jax.ShapeDtypeStruct((REL_PAD,), f32)] * 3
        + [jax.ShapeDtypeStruct((REL_PAD,), i32)],
        mesh=plsc.VectorSubcoreMesh(core_axis_name="c", subcore_axis_name="s"),
        scratch_types=[
            pltpu.VMEM((OBJ_PAD,), f32),
            pltpu.VMEM((REL_W,), i32),
            pltpu.VMEM((REL_W,), i32),
            pltpu.VMEM((REL_W,), f32),
            pltpu.VMEM((REL_W,), f32),
            pltpu.VMEM((REL_W,), f32),
            pltpu.VMEM((REL_W,), f32),
            pltpu.VMEM((REL_W,), f32),
            pltpu.VMEM((REL_W,), f32),
            pltpu.VMEM((REL_W,), i32),
        ],
        compiler_params=pltpu.CompilerParams(needs_layout_passes=False),
    )
    return kfn(obj_scores, idx0, idx1, s1, s2, s3)


ROWS = N_SORT // 128


def _partner(x, j, row, lane):
    if j >= 128:
        r = j // 128
        a = jnp.concatenate([x[r:], x[:r]], axis=0)
        b = jnp.concatenate([x[-r:], x[:-r]], axis=0)
        return jnp.where((row & r) == 0, a, b)
    a = jnp.concatenate([x[:, j:], x[:, :j]], axis=1)
    b = jnp.concatenate([x[:, -j:], x[:, :-j]], axis=1)
    return jnp.where((lane & j) == 0, a, b)


def _sort_body(key_ref, key_o, idx_o, mod_o):
    row = lax.broadcasted_iota(jnp.int32, (ROWS, 128), 0)
    lane = lax.broadcasted_iota(jnp.int32, (ROWS, 128), 1)
    e = row * 128 + lane
    key = key_ref[...]
    idx = e
    kk = 2
    while kk <= N_SORT:
        desc = (e & kk) == 0
        j = kk // 2
        while j >= 1:
            pk = _partner(key, j, row, lane)
            pi = _partner(idx, j, row, lane)
            if j >= 128:
                is_low = (row & (j // 128)) == 0
            else:
                is_low = (lane & j) == 0
            a_wins = (key > pk) | ((key == pk) & (idx < pi))
            take_a = a_wins == (is_low == desc)
            key = jnp.where(take_a, key, pk)
            idx = jnp.where(take_a, idx, pi)
            j //= 2
        kk *= 2
    key_o[...] = key
    idx_o[...] = idx
    mod_o[...] = idx % NUM_REL


def _sort_call(keys_pad):
    return pl.pallas_call(
        _sort_body,
        out_shape=[jax.ShapeDtypeStruct((ROWS, 128), jnp.float32),
                   jax.ShapeDtypeStruct((ROWS, 128), jnp.int32),
                   jax.ShapeDtypeStruct((ROWS, 128), jnp.int32)],
    )(keys_pad)


ROWS_W = N_SORT // NW
CHUNK = 512


def _gath_body(rcat_hbm, m_hbm, ord_hbm, pp_hbm, pc_hbm,
               probs_hbm, o0_hbm, o1_hbm, oc_hbm,
               tp_v, tc_v, mc_v, ord_v, rows_v, o0_v, o1_v, oc_v, sem):
    wid = lax.axis_index("s") * 2 + lax.axis_index("c")
    pltpu.sync_copy(pp_hbm, tp_v)
    pltpu.sync_copy(pc_hbm, tc_v)
    nch = ROWS_W // CHUNK
    for c in range(nch):
        r = wid * nch + c
        pltpu.sync_copy(m_hbm.at[r], mc_v)
        pltpu.sync_copy(ord_hbm.at[r], ord_v)
        d = pltpu.async_copy(rcat_hbm.at[mc_v], rows_v, sem)
        for v in range(CHUNK // 16):
            sl = pl.ds(v * 16, 16)
            mv = mc_v[sl]
            ov = ord_v[sl]
            pp = plsc.load_gather(tp_v, [mv])
            pc = plsc.load_gather(tc_v, [mv])
            o0_v[sl] = pp >> 13
            o1_v[sl] = pp & 8191
            c1 = pc & 63
            c2 = (pc >> 6) & 63
            c3 = (pc >> 12) & 63
            oc_v[sl] = jnp.where(ov < NUM_REL, c1,
                                 jnp.where(ov < 2 * NUM_REL, c2, c3))
        d.wait()
        base = r * CHUNK
        pltpu.sync_copy(rows_v, probs_hbm.at[pl.ds(base, CHUNK)])
        pltpu.sync_copy(o0_v, o0_hbm.at[pl.ds(base, CHUNK)])
        pltpu.sync_copy(o1_v, o1_hbm.at[pl.ds(base, CHUNK)])
        pltpu.sync_copy(oc_v, oc_hbm.at[pl.ds(base, CHUNK)])


def _gath_call(rcat, m2, ord2, pp, pc):
    i32 = jnp.int32
    kfn = pl.kernel(
        _gath_body,
        out_type=[jax.ShapeDtypeStruct((N_SORT, 128), jnp.float32),
                  jax.ShapeDtypeStruct((N_SORT,), i32),
                  jax.ShapeDtypeStruct((N_SORT,), i32),
                  jax.ShapeDtypeStruct((N_SORT,), i32)],
        mesh=plsc.VectorSubcoreMesh(core_axis_name="c", subcore_axis_name="s"),
        scratch_types=[
            pltpu.VMEM((REL_PAD,), i32),
            pltpu.VMEM((REL_PAD,), i32),
            pltpu.VMEM((CHUNK,), i32),
            pltpu.VMEM((CHUNK,), i32),
            pltpu.VMEM((CHUNK, 128), jnp.float32),
            pltpu.VMEM((CHUNK,), i32),
            pltpu.VMEM((CHUNK,), i32),
            pltpu.VMEM((CHUNK,), i32),
            pltpu.SemaphoreType.DMA,
        ],
        compiler_params=pltpu.CompilerParams(needs_layout_passes=False),
    )
    return kfn(rcat, m2, ord2, pp, pc)


def kernel(rel1_prob, rel2_prob, rel3_prob, super_rel_prob, refine_logits,
           rel_pair_idx, boxes):
    pair_dtype = rel_pair_idx.dtype
    pair = rel_pair_idx.astype(jnp.int32)

    sc8, pr8 = _obj_call(refine_logits.T)
    obj_scores = sc8[0]
    obj_pred = pr8[0]
    obj_scores_pad = jnp.concatenate(
        [obj_scores, jnp.zeros((OBJ_PAD - NUM_OBJ,), jnp.float32)])

    s8, c8 = _rel_call(rel1_prob.T, rel2_prob.T, rel3_prob.T)
    rcat, = _rcat_call(rel1_prob, rel2_prob, rel3_prob)

    zpad_i = jnp.zeros((REL_PAD - NUM_REL,), jnp.int32)
    zpad_f = jnp.zeros((REL_PAD - NUM_REL,), jnp.float32)
    idx0 = jnp.concatenate([pair[:, 0], zpad_i])
    idx1 = jnp.concatenate([pair[:, 1], zpad_i])
    s1p = jnp.concatenate([s8[0], zpad_f])
    s2p = jnp.concatenate([s8[1], zpad_f])
    s3p = jnp.concatenate([s8[2], zpad_f])

    pc_pad = jnp.concatenate([c8[0], zpad_i])
    k1, k2, k3, pp = _keys_call(obj_scores_pad, idx0, idx1, s1p, s2p, s3p)

    neg = jnp.full((N_SORT - N_CAT,), -jnp.inf, jnp.float32)
    keys_pad = jnp.concatenate(
        [k1[:NUM_REL], k2[:NUM_REL], k3[:NUM_REL], neg]).reshape(ROWS, 128)

    skey, sidx, smod = _sort_call(keys_pad)
    order = sidx.reshape(N_SORT)
    m = smod.reshape(N_SORT)

    probs, o0, o1, oc = _gath_call(
        rcat, m.reshape(N_SORT // CHUNK, CHUNK),
        order.reshape(N_SORT // CHUNK, CHUNK), pp, pc_pad)

    triple_scores_sorted = skey.reshape(N_SORT)[:N_CAT]
    rel_class_sorted = oc[:N_CAT]
    rel_pair_sorted = jnp.stack([o0[:N_CAT], o1[:N_CAT]],
                                axis=1).astype(pair_dtype)
    class_prob_sorted = probs[:N_CAT, :50]
    return (triple_scores_sorted, rel_class_sorted, rel_pair_sorted,
            class_prob_sorted, obj_pred, obj_scores)


# speedup vs baseline: 1.0478x; 1.0478x over previous
import functools

import jax
import jax.numpy as jnp
from jax import lax
from jax.experimental import pallas as pl
from jax.experimental.pallas import tpu as pltpu
from jax.experimental.pallas import tpu_sc as plsc

GEO = [1, 2, 3, 4, 5, 6, 8, 10, 22, 23, 29, 31, 32, 33, 43]
POS = [9, 16, 17, 20, 27, 30, 36, 42, 48, 49, 50]
SEM = [7, 11, 12, 13, 14, 15, 18, 19, 21, 24, 25, 26, 28, 34, 35, 37, 38,
       39, 40, 41, 44, 45, 46, 47]

NUM_OBJ = 5000
OBJ_PAD = 5120
NUM_REL = 20000
NUM_CLS = 151
N_CAT = 3 * NUM_REL
CAT_PAD = 61440
N_SORT = 65536
REL_PAD = 20480
NW = 32
REL_W = REL_PAD // NW


def _obj_body(logits_ref, score_ref, pred_ref):
    x = logits_ref[...]
    cc = x.shape[1]
    m = jnp.max(x, axis=0, keepdims=True)
    e = jnp.exp(x - m)
    t = jnp.concatenate([e, jnp.zeros((1, cc), jnp.float32)], axis=0)
    acc = t[0:8]
    for g in range(1, 19):
        acc = acc + t[8 * g:8 * g + 8]
    acc = acc[0:4] + acc[4:8]
    acc = acc[0:2] + acc[2:4]
    s = acc[0:1] + acc[1:2]
    p = e / s
    rowi = lax.broadcasted_iota(jnp.int32, x.shape, 0)
    p = jnp.where(rowi == 0, -1.0, p)
    mx = jnp.max(p, axis=0, keepdims=True)
    am = jnp.min(jnp.where(p == mx, rowi, NUM_CLS + 1), axis=0, keepdims=True)
    score_ref[...] = jnp.broadcast_to(mx, (8, cc))
    pred_ref[...] = jnp.broadcast_to(am, (8, cc))


def _obj_call(refine_logits_t):
    return pl.pallas_call(
        _obj_body,
        out_shape=[jax.ShapeDtypeStruct((8, NUM_OBJ), jnp.float32),
                   jax.ShapeDtypeStruct((8, NUM_OBJ), jnp.int32)],
    )(refine_logits_t)


def _branch_t(e, labels):
    mx = jnp.max(e, axis=0, keepdims=True)
    rowi = lax.broadcasted_iota(jnp.int32, e.shape, 0)
    am = jnp.min(jnp.where(e == mx, rowi, 1000), axis=0, keepdims=True)
    cls = jnp.full_like(am, labels[0])
    for k in range(1, len(labels)):
        cls = jnp.where(am == k, labels[k], cls)
    return mx, cls


def _rel_body(r1_ref, r2_ref, r3_ref, s_ref, c_ref):
    e1 = jnp.exp(r1_ref[...])
    e2 = jnp.exp(r2_ref[...])
    e3 = jnp.exp(r3_ref[...])
    s1, c1 = _branch_t(e1, GEO)
    s2, c2 = _branch_t(e2, POS)
    s3, c3 = _branch_t(e3, SEM)
    pc = c1 + c2 * 64 + c3 * 4096
    s_ref[...] = jnp.concatenate([s1, s2, s3, s3, s3, s3, s3, s3], axis=0)
    c_ref[...] = jnp.concatenate([pc, pc, pc, pc, pc, pc, pc, pc], axis=0)


def _rel_call(r1t, r2t, r3t):
    return pl.pallas_call(
        _rel_body,
        out_shape=[jax.ShapeDtypeStruct((8, NUM_REL), jnp.float32),
                   jax.ShapeDtypeStruct((8, NUM_REL), jnp.int32)],
    )(r1t, r2t, r3t)


def _rcat_body(r1_ref, r2_ref, r3_ref, rcat_ref):
    e1 = jnp.exp(r1_ref[...])
    e2 = jnp.exp(r2_ref[...])
    e3 = jnp.exp(r3_ref[...])
    pad = jnp.zeros((e1.shape[0], 78), jnp.float32)
    rcat_ref[...] = jnp.concatenate([e1, e2, e3, pad], axis=1)


def _rcat_call(r1, r2, r3):
    c = 2000
    return pl.pallas_call(
        _rcat_body,
        grid=(NUM_REL // c,),
        in_specs=[pl.BlockSpec((c, 15), lambda i: (i, 0)),
                  pl.BlockSpec((c, 11), lambda i: (i, 0)),
                  pl.BlockSpec((c, 24), lambda i: (i, 0))],
        out_specs=[pl.BlockSpec((c, 128), lambda i: (i, 0))],
        out_shape=[jax.ShapeDtypeStruct((NUM_REL, 128), jnp.float32)],
    )(r1, r2, r3)


def _keys_body(obj_hbm, idx0_hbm, idx1_hbm, s1_hbm, s2_hbm, s3_hbm,
               k1_hbm, k2_hbm, k3_hbm, pp_hbm,
               table_v, idx0_v, idx1_v, s1_v, s2_v, s3_v, k1_v, k2_v, k3_v,
               pp_v):
    wid = lax.axis_index("s") * 2 + lax.axis_index("c")
    base = wid * REL_W
    pltpu.sync_copy(obj_hbm, table_v)
    pltpu.sync_copy(idx0_hbm.at[pl.ds(base, REL_W)], idx0_v)
    pltpu.sync_copy(idx1_hbm.at[pl.ds(base, REL_W)], idx1_v)
    pltpu.sync_copy(s1_hbm.at[pl.ds(base, REL_W)], s1_v)
    pltpu.sync_copy(s2_hbm.at[pl.ds(base, REL_W)], s2_v)
    pltpu.sync_copy(s3_hbm.at[pl.ds(base, REL_W)], s3_v)
    for v in range(REL_W // 16):
        sl = pl.ds(v * 16, 16)
        i0 = idx0_v[sl]
        i1 = idx1_v[sl]
        g0 = plsc.load_gather(table_v, [i0])
        g1 = plsc.load_gather(table_v, [i1])
        k1_v[sl] = (s1_v[sl] * g0) * g1
        k2_v[sl] = (s2_v[sl] * g0) * g1
        k3_v[sl] = (s3_v[sl] * g0) * g1
        pp_v[sl] = i0 * 8192 + i1
    pltpu.sync_copy(k1_v, k1_hbm.at[pl.ds(base, REL_W)])
    pltpu.sync_copy(k2_v, k2_hbm.at[pl.ds(base, REL_W)])
    pltpu.sync_copy(k3_v, k3_hbm.at[pl.ds(base, REL_W)])
    pltpu.sync_copy(pp_v, pp_hbm.at[pl.ds(base, REL_W)])


def _keys_call(obj_scores, idx0, idx1, s1, s2, s3):
    f32 = jnp.float32
    i32 = jnp.int32
    kfn = pl.kernel(
        _keys_body,
        out_type=[jax.ShapeDtypeStruct((REL_PAD,), f32)] * 3
        + [jax.ShapeDtypeStruct((REL_PAD,), i32)],
        mesh=plsc.VectorSubcoreMesh(core_axis_name="c", subcore_axis_name="s"),
        scratch_types=[
            pltpu.VMEM((OBJ_PAD,), f32),
            pltpu.VMEM((REL_W,), i32),
            pltpu.VMEM((REL_W,), i32),
            pltpu.VMEM((REL_W,), f32),
            pltpu.VMEM((REL_W,), f32),
            pltpu.VMEM((REL_W,), f32),
            pltpu.VMEM((REL_W,), f32),
            pltpu.VMEM((REL_W,), f32),
            pltpu.VMEM((REL_W,), f32),
            pltpu.VMEM((REL_W,), i32),
        ],
        compiler_params=pltpu.CompilerParams(needs_layout_passes=False),
    )
    return kfn(obj_scores, idx0, idx1, s1, s2, s3)


ROWS = N_SORT // 128


def _partner(x, j, row, lane):
    if j >= 128:
        r = j // 128
        a = jnp.concatenate([x[r:], x[:r]], axis=0)
        b = jnp.concatenate([x[-r:], x[:-r]], axis=0)
        return jnp.where((row & r) == 0, a, b)
    a = jnp.concatenate([x[:, j:], x[:, :j]], axis=1)
    b = jnp.concatenate([x[:, -j:], x[:, :-j]], axis=1)
    return jnp.where((lane & j) == 0, a, b)


def _sort_body(key_ref, key_o, idx_o, mod_o):
    row = lax.broadcasted_iota(jnp.int32, (ROWS, 128), 0)
    lane = lax.broadcasted_iota(jnp.int32, (ROWS, 128), 1)
    e = row * 128 + lane
    key = key_ref[...]
    idx = e
    kk = 2
    while kk <= N_SORT:
        desc = (e & kk) == 0
        j = kk // 2
        while j >= 1:
            pk = _partner(key, j, row, lane)
            pi = _partner(idx, j, row, lane)
            if j >= 128:
                is_low = (row & (j // 128)) == 0
            else:
                is_low = (lane & j) == 0
            a_wins = (key > pk) | ((key == pk) & (idx < pi))
            take_a = a_wins == (is_low == desc)
            key = jnp.where(take_a, key, pk)
            idx = jnp.where(take_a, idx, pi)
            j //= 2
        kk *= 2
    key_o[...] = key
    idx_o[...] = idx
    mod_o[...] = idx % NUM_REL


def _sort_call(keys_pad):
    return pl.pallas_call(
        _sort_body,
        out_shape=[jax.ShapeDtypeStruct((ROWS, 128), jnp.float32),
                   jax.ShapeDtypeStruct((ROWS, 128), jnp.int32),
                   jax.ShapeDtypeStruct((ROWS, 128), jnp.int32)],
    )(keys_pad)


ROWS_W = N_SORT // NW
CHUNK = 512


def _probs_body(rcat_hbm, m_hbm, out_hbm, mc_v, rows_v, sem):
    wid = lax.axis_index("s") * 2 + lax.axis_index("c")
    for c in range(ROWS_W // CHUNK):
        r = wid * (ROWS_W // CHUNK) + c
        pltpu.sync_copy(m_hbm.at[r], mc_v)
        pltpu.async_copy(rcat_hbm.at[mc_v], rows_v, sem).wait()
        pltpu.sync_copy(rows_v, out_hbm.at[pl.ds(r * CHUNK, CHUNK)])


def _probs_call(rcat, m2):
    kfn = pl.kernel(
        _probs_body,
        out_type=jax.ShapeDtypeStruct((N_SORT, 128), jnp.float32),
        mesh=plsc.VectorSubcoreMesh(core_axis_name="c", subcore_axis_name="s"),
        scratch_types=[
            pltpu.VMEM((CHUNK,), jnp.int32),
            pltpu.VMEM((CHUNK, 128), jnp.float32),
            pltpu.SemaphoreType.DMA,
        ],
        compiler_params=pltpu.CompilerParams(needs_layout_passes=False),
    )
    return kfn(rcat, m2)


def _elem_body(pp_hbm, pc_hbm, ord_hbm, m_hbm,
               o0_hbm, o1_hbm, oc_hbm,
               tp_v, tc_v, ord_v, m_v, o0_v, o1_v, oc_v):
    wid = lax.axis_index("s") * 2 + lax.axis_index("c")
    base = wid * ROWS_W
    pltpu.sync_copy(pp_hbm, tp_v)
    pltpu.sync_copy(pc_hbm, tc_v)
    pltpu.sync_copy(ord_hbm.at[pl.ds(base, ROWS_W)], ord_v)
    pltpu.sync_copy(m_hbm.at[pl.ds(base, ROWS_W)], m_v)

    def step(v, _):
        sl = pl.ds(v * 16, 16)
        mv = m_v[sl]
        ov = ord_v[sl]
        pp = plsc.load_gather(tp_v, [mv])
        pc = plsc.load_gather(tc_v, [mv])
        o0_v[sl] = pp >> 13
        o1_v[sl] = pp & 8191
        c1 = pc & 63
        c2 = (pc >> 6) & 63
        c3 = (pc >> 12) & 63
        oc_v[sl] = jnp.where(ov < NUM_REL, c1,
                             jnp.where(ov < 2 * NUM_REL, c2, c3))
        return _

    lax.fori_loop(0, ROWS_W // 16, step, 0)
    pltpu.sync_copy(o0_v, o0_hbm.at[pl.ds(base, ROWS_W)])
    pltpu.sync_copy(o1_v, o1_hbm.at[pl.ds(base, ROWS_W)])
    pltpu.sync_copy(oc_v, oc_hbm.at[pl.ds(base, ROWS_W)])


def _elem_call(pp, pc, order, m):
    i32 = jnp.int32
    kfn = pl.kernel(
        _elem_body,
        out_type=[jax.ShapeDtypeStruct((N_SORT,), i32)] * 3,
        mesh=plsc.VectorSubcoreMesh(core_axis_name="c", subcore_axis_name="s"),
        scratch_types=[
            pltpu.VMEM((REL_PAD,), i32),
            pltpu.VMEM((REL_PAD,), i32),
            pltpu.VMEM((ROWS_W,), i32),
            pltpu.VMEM((ROWS_W,), i32),
            pltpu.VMEM((ROWS_W,), i32),
            pltpu.VMEM((ROWS_W,), i32),
            pltpu.VMEM((ROWS_W,), i32),
        ],
        compiler_params=pltpu.CompilerParams(needs_layout_passes=False),
    )
    return kfn(pp, pc, order, m)


def kernel(rel1_prob, rel2_prob, rel3_prob, super_rel_prob, refine_logits,
           rel_pair_idx, boxes):
    pair_dtype = rel_pair_idx.dtype
    pair = rel_pair_idx.astype(jnp.int32)

    sc8, pr8 = _obj_call(refine_logits.T)
    obj_scores = sc8[0]
    obj_pred = pr8[0]
    obj_scores_pad = jnp.concatenate(
        [obj_scores, jnp.zeros((OBJ_PAD - NUM_OBJ,), jnp.float32)])

    s8, c8 = _rel_call(rel1_prob.T, rel2_prob.T, rel3_prob.T)
    rcat, = _rcat_call(rel1_prob, rel2_prob, rel3_prob)

    zpad_i = jnp.zeros((REL_PAD - NUM_REL,), jnp.int32)
    zpad_f = jnp.zeros((REL_PAD - NUM_REL,), jnp.float32)
    idx0 = jnp.concatenate([pair[:, 0], zpad_i])
    idx1 = jnp.concatenate([pair[:, 1], zpad_i])
    s1p = jnp.concatenate([s8[0], zpad_f])
    s2p = jnp.concatenate([s8[1], zpad_f])
    s3p = jnp.concatenate([s8[2], zpad_f])

    pc_pad = jnp.concatenate([c8[0], zpad_i])
    k1, k2, k3, pp = _keys_call(obj_scores_pad, idx0, idx1, s1p, s2p, s3p)

    neg = jnp.full((N_SORT - N_CAT,), -jnp.inf, jnp.float32)
    keys_pad = jnp.concatenate(
        [k1[:NUM_REL], k2[:NUM_REL], k3[:NUM_REL], neg]).reshape(ROWS, 128)

    skey, sidx, smod = _sort_call(keys_pad)
    order = sidx.reshape(N_SORT)
    m = smod.reshape(N_SORT)

    probs = _probs_call(rcat, m.reshape(N_SORT // CHUNK, CHUNK))
    o0, o1, oc = _elem_call(pp, pc_pad, order, m)

    triple_scores_sorted = skey.reshape(N_SORT)[:N_CAT]
    rel_class_sorted = oc[:N_CAT]
    rel_pair_sorted = jnp.stack([o0[:N_CAT], o1[:N_CAT]],
                                axis=1).astype(pair_dtype)
    class_prob_sorted = probs[:N_CAT, :50]
    return (triple_scores_sorted, rel_class_sorted, rel_pair_sorted,
            class_prob_sorted, obj_pred, obj_scores)


# speedup vs baseline: 1.1112x; 1.0605x over previous
import functools

import jax
import jax.numpy as jnp
from jax import lax
from jax.experimental import pallas as pl
from jax.experimental.pallas import tpu as pltpu
from jax.experimental.pallas import tpu_sc as plsc

GEO = [1, 2, 3, 4, 5, 6, 8, 10, 22, 23, 29, 31, 32, 33, 43]
POS = [9, 16, 17, 20, 27, 30, 36, 42, 48, 49, 50]
SEM = [7, 11, 12, 13, 14, 15, 18, 19, 21, 24, 25, 26, 28, 34, 35, 37, 38,
       39, 40, 41, 44, 45, 46, 47]

NUM_OBJ = 5000
OBJ_PAD = 5120
NUM_REL = 20000
NUM_CLS = 151
N_CAT = 3 * NUM_REL
CAT_PAD = 61440
N_SORT = 65536
REL_PAD = 20480
NW = 32
REL_W = REL_PAD // NW


def _obj_body(logits_ref, score_ref, pred_ref):
    x = logits_ref[...]
    cc = x.shape[1]
    m = jnp.max(x, axis=0, keepdims=True)
    e = jnp.exp(x - m)
    t = jnp.concatenate([e, jnp.zeros((1, cc), jnp.float32)], axis=0)
    acc = t[0:8]
    for g in range(1, 19):
        acc = acc + t[8 * g:8 * g + 8]
    acc = acc[0:4] + acc[4:8]
    acc = acc[0:2] + acc[2:4]
    s = acc[0:1] + acc[1:2]
    p = e / s
    rowi = lax.broadcasted_iota(jnp.int32, x.shape, 0)
    p = jnp.where(rowi == 0, -1.0, p)
    mx = jnp.max(p, axis=0, keepdims=True)
    am = jnp.min(jnp.where(p == mx, rowi, NUM_CLS + 1), axis=0, keepdims=True)
    score_ref[...] = jnp.broadcast_to(mx, (8, cc))
    pred_ref[...] = jnp.broadcast_to(am, (8, cc))


def _obj_call(refine_logits_t):
    return pl.pallas_call(
        _obj_body,
        out_shape=[jax.ShapeDtypeStruct((8, NUM_OBJ), jnp.float32),
                   jax.ShapeDtypeStruct((8, NUM_OBJ), jnp.int32)],
    )(refine_logits_t)


def _branch_t(e, labels):
    mx = jnp.max(e, axis=0, keepdims=True)
    rowi = lax.broadcasted_iota(jnp.int32, e.shape, 0)
    am = jnp.min(jnp.where(e == mx, rowi, 1000), axis=0, keepdims=True)
    cls = jnp.full_like(am, labels[0])
    for k in range(1, len(labels)):
        cls = jnp.where(am == k, labels[k], cls)
    return mx, cls


def _rel_body(r1_ref, r2_ref, r3_ref, s_ref, c_ref):
    e1 = jnp.exp(r1_ref[...])
    e2 = jnp.exp(r2_ref[...])
    e3 = jnp.exp(r3_ref[...])
    s1, c1 = _branch_t(e1, GEO)
    s2, c2 = _branch_t(e2, POS)
    s3, c3 = _branch_t(e3, SEM)
    pc = c1 + c2 * 64 + c3 * 4096
    s_ref[...] = jnp.concatenate([s1, s2, s3, s3, s3, s3, s3, s3], axis=0)
    c_ref[...] = jnp.concatenate([pc, pc, pc, pc, pc, pc, pc, pc], axis=0)


def _rel_call(r1t, r2t, r3t):
    return pl.pallas_call(
        _rel_body,
        out_shape=[jax.ShapeDtypeStruct((8, NUM_REL), jnp.float32),
                   jax.ShapeDtypeStruct((8, NUM_REL), jnp.int32)],
    )(r1t, r2t, r3t)


def _rcat_body(r1_ref, r2_ref, r3_ref, rcat_ref):
    e1 = jnp.exp(r1_ref[...])
    e2 = jnp.exp(r2_ref[...])
    e3 = jnp.exp(r3_ref[...])
    pad = jnp.zeros((e1.shape[0], 78), jnp.float32)
    rcat_ref[...] = jnp.concatenate([e1, e2, e3, pad], axis=1)


def _rcat_call(r1, r2, r3):
    c = 2000
    return pl.pallas_call(
        _rcat_body,
        grid=(NUM_REL // c,),
        in_specs=[pl.BlockSpec((c, 15), lambda i: (i, 0)),
                  pl.BlockSpec((c, 11), lambda i: (i, 0)),
                  pl.BlockSpec((c, 24), lambda i: (i, 0))],
        out_specs=[pl.BlockSpec((c, 128), lambda i: (i, 0))],
        out_shape=[jax.ShapeDtypeStruct((NUM_REL, 128), jnp.float32)],
    )(r1, r2, r3)


def _keys_body(obj_hbm, idx0_hbm, idx1_hbm, s1_hbm, s2_hbm, s3_hbm,
               k1_hbm, k2_hbm, k3_hbm, pp_hbm,
               table_v, idx0_v, idx1_v, s1_v, s2_v, s3_v, k1_v, k2_v, k3_v,
               pp_v):
    wid = lax.axis_index("s") * 2 + lax.axis_index("c")
    base = wid * REL_W
    pltpu.sync_copy(obj_hbm, table_v)
    pltpu.sync_copy(idx0_hbm.at[pl.ds(base, REL_W)], idx0_v)
    pltpu.sync_copy(idx1_hbm.at[pl.ds(base, REL_W)], idx1_v)
    pltpu.sync_copy(s1_hbm.at[pl.ds(base, REL_W)], s1_v)
    pltpu.sync_copy(s2_hbm.at[pl.ds(base, REL_W)], s2_v)
    pltpu.sync_copy(s3_hbm.at[pl.ds(base, REL_W)], s3_v)
    for v in range(REL_W // 16):
        sl = pl.ds(v * 16, 16)
        i0 = idx0_v[sl]
        i1 = idx1_v[sl]
        g0 = plsc.load_gather(table_v, [i0])
        g1 = plsc.load_gather(table_v, [i1])
        k1_v[sl] = (s1_v[sl] * g0) * g1
        k2_v[sl] = (s2_v[sl] * g0) * g1
        k3_v[sl] = (s3_v[sl] * g0) * g1
        pp_v[sl] = i0 * 8192 + i1
    pltpu.sync_copy(k1_v, k1_hbm.at[pl.ds(base, REL_W)])
    pltpu.sync_copy(k2_v, k2_hbm.at[pl.ds(base, REL_W)])
    pltpu.sync_copy(k3_v, k3_hbm.at[pl.ds(base, REL_W)])
    pltpu.sync_copy(pp_v, pp_hbm.at[pl.ds(base, REL_W)])


def _keys_call(obj_scores, idx0, idx1, s1, s2, s3):
    f32 = jnp.float32
    i32 = jnp.int32
    kfn = pl.kernel(
        _keys_body,
        out_type=[jax.ShapeDtypeStruct((REL_PAD,), f32)] * 3
        + [jax.ShapeDtypeStruct((REL_PAD,), i32)],
        mesh=plsc.VectorSubcoreMesh(core_axis_name="c", subcore_axis_name="s"),
        scratch_types=[
            pltpu.VMEM((OBJ_PAD,), f32),
            pltpu.VMEM((REL_W,), i32),
            pltpu.VMEM((REL_W,), i32),
            pltpu.VMEM((REL_W,), f32),
            pltpu.VMEM((REL_W,), f32),
            pltpu.VMEM((REL_W,), f32),
            pltpu.VMEM((REL_W,), f32),
            pltpu.VMEM((REL_W,), f32),
            pltpu.VMEM((REL_W,), f32),
            pltpu.VMEM((REL_W,), i32),
        ],
        compiler_params=pltpu.CompilerParams(needs_layout_passes=False),
    )
    return kfn(obj_scores, idx0, idx1, s1, s2, s3)


ROWS = N_SORT // 128


def _partner(x, j, row, lane):
    if j < ROWS:
        a = jnp.concatenate([x[j:], x[:j]], axis=0)
        b = jnp.concatenate([x[-j:], x[:-j]], axis=0)
        return jnp.where((row & j) == 0, a, b)
    jj = j // ROWS
    a = jnp.concatenate([x[:, jj:], x[:, :jj]], axis=1)
    b = jnp.concatenate([x[:, -jj:], x[:, :-jj]], axis=1)
    return jnp.where((lane & jj) == 0, a, b)


def _sort_body(key_ref, key_o, idx_o, mod_o):
    row = lax.broadcasted_iota(jnp.int32, (ROWS, 128), 0)
    lane = lax.broadcasted_iota(jnp.int32, (ROWS, 128), 1)
    key = key_ref[...]
    idx = lane * ROWS + row
    kk = 2
    while kk <= N_SORT:
        if kk < ROWS:
            desc = (row & kk) == 0
        else:
            desc = (lane & (kk // ROWS)) == 0
        j = kk // 2
        while j >= 1:
            pk = _partner(key, j, row, lane)
            pi = _partner(idx, j, row, lane)
            if j < ROWS:
                is_low = (row & j) == 0
            else:
                is_low = (lane & (j // ROWS)) == 0
            a_wins = (key > pk) | ((key == pk) & (idx < pi))
            take_a = a_wins == (is_low == desc)
            key = jnp.where(take_a, key, pk)
            idx = jnp.where(take_a, idx, pi)
            j //= 2
        kk *= 2
    key_o[...] = key
    idx_o[...] = idx
    mod_o[...] = idx % NUM_REL


def _sort_call(keys_pad):
    return pl.pallas_call(
        _sort_body,
        out_shape=[jax.ShapeDtypeStruct((ROWS, 128), jnp.float32),
                   jax.ShapeDtypeStruct((ROWS, 128), jnp.int32),
                   jax.ShapeDtypeStruct((ROWS, 128), jnp.int32)],
    )(keys_pad)


ROWS_W = N_SORT // NW
CHUNK = 512


def _probs_body(rcat_hbm, m_hbm, out_hbm, mc_v, rows_v, sem):
    wid = lax.axis_index("s") * 2 + lax.axis_index("c")
    for c in range(ROWS_W // CHUNK):
        r = wid * (ROWS_W // CHUNK) + c
        pltpu.sync_copy(m_hbm.at[r], mc_v)
        pltpu.async_copy(rcat_hbm.at[mc_v], rows_v, sem).wait()
        pltpu.sync_copy(rows_v, out_hbm.at[pl.ds(r * CHUNK, CHUNK)])


def _probs_call(rcat, m2):
    kfn = pl.kernel(
        _probs_body,
        out_type=jax.ShapeDtypeStruct((N_SORT, 128), jnp.float32),
        mesh=plsc.VectorSubcoreMesh(core_axis_name="c", subcore_axis_name="s"),
        scratch_types=[
            pltpu.VMEM((CHUNK,), jnp.int32),
            pltpu.VMEM((CHUNK, 128), jnp.float32),
            pltpu.SemaphoreType.DMA,
        ],
        compiler_params=pltpu.CompilerParams(needs_layout_passes=False),
    )
    return kfn(rcat, m2)


def _elem_body(pp_hbm, pc_hbm, ord_hbm, m_hbm,
               o0_hbm, o1_hbm, oc_hbm,
               tp_v, tc_v, ord_v, m_v, o0_v, o1_v, oc_v):
    wid = lax.axis_index("s") * 2 + lax.axis_index("c")
    base = wid * ROWS_W
    pltpu.sync_copy(pp_hbm, tp_v)
    pltpu.sync_copy(pc_hbm, tc_v)
    pltpu.sync_copy(ord_hbm.at[pl.ds(base, ROWS_W)], ord_v)
    pltpu.sync_copy(m_hbm.at[pl.ds(base, ROWS_W)], m_v)

    def step(v, _):
        sl = pl.ds(v * 16, 16)
        mv = m_v[sl]
        ov = ord_v[sl]
        pp = plsc.load_gather(tp_v, [mv])
        pc = plsc.load_gather(tc_v, [mv])
        o0_v[sl] = pp >> 13
        o1_v[sl] = pp & 8191
        c1 = pc & 63
        c2 = (pc >> 6) & 63
        c3 = (pc >> 12) & 63
        oc_v[sl] = jnp.where(ov < NUM_REL, c1,
                             jnp.where(ov < 2 * NUM_REL, c2, c3))
        return _

    lax.fori_loop(0, ROWS_W // 16, step, 0)
    pltpu.sync_copy(o0_v, o0_hbm.at[pl.ds(base, ROWS_W)])
    pltpu.sync_copy(o1_v, o1_hbm.at[pl.ds(base, ROWS_W)])
    pltpu.sync_copy(oc_v, oc_hbm.at[pl.ds(base, ROWS_W)])


def _elem_call(pp, pc, order, m):
    i32 = jnp.int32
    kfn = pl.kernel(
        _elem_body,
        out_type=[jax.ShapeDtypeStruct((N_SORT,), i32)] * 3,
        mesh=plsc.VectorSubcoreMesh(core_axis_name="c", subcore_axis_name="s"),
        scratch_types=[
            pltpu.VMEM((REL_PAD,), i32),
            pltpu.VMEM((REL_PAD,), i32),
            pltpu.VMEM((ROWS_W,), i32),
            pltpu.VMEM((ROWS_W,), i32),
            pltpu.VMEM((ROWS_W,), i32),
            pltpu.VMEM((ROWS_W,), i32),
            pltpu.VMEM((ROWS_W,), i32),
        ],
        compiler_params=pltpu.CompilerParams(needs_layout_passes=False),
    )
    return kfn(pp, pc, order, m)


def kernel(rel1_prob, rel2_prob, rel3_prob, super_rel_prob, refine_logits,
           rel_pair_idx, boxes):
    pair_dtype = rel_pair_idx.dtype
    pair = rel_pair_idx.astype(jnp.int32)

    sc8, pr8 = _obj_call(refine_logits.T)
    obj_scores = sc8[0]
    obj_pred = pr8[0]
    obj_scores_pad = jnp.concatenate(
        [obj_scores, jnp.zeros((OBJ_PAD - NUM_OBJ,), jnp.float32)])

    s8, c8 = _rel_call(rel1_prob.T, rel2_prob.T, rel3_prob.T)
    rcat, = _rcat_call(rel1_prob, rel2_prob, rel3_prob)

    zpad_i = jnp.zeros((REL_PAD - NUM_REL,), jnp.int32)
    zpad_f = jnp.zeros((REL_PAD - NUM_REL,), jnp.float32)
    idx0 = jnp.concatenate([pair[:, 0], zpad_i])
    idx1 = jnp.concatenate([pair[:, 1], zpad_i])
    s1p = jnp.concatenate([s8[0], zpad_f])
    s2p = jnp.concatenate([s8[1], zpad_f])
    s3p = jnp.concatenate([s8[2], zpad_f])

    pc_pad = jnp.concatenate([c8[0], zpad_i])
    k1, k2, k3, pp = _keys_call(obj_scores_pad, idx0, idx1, s1p, s2p, s3p)

    neg = jnp.full((N_SORT - N_CAT,), -jnp.inf, jnp.float32)
    keys_pad = jnp.concatenate(
        [k1[:NUM_REL], k2[:NUM_REL], k3[:NUM_REL], neg]).reshape(128, ROWS).T

    skey, sidx, smod = _sort_call(keys_pad)
    order = sidx.T.reshape(N_SORT)
    m = smod.T.reshape(N_SORT)

    probs = _probs_call(rcat, m.reshape(N_SORT // CHUNK, CHUNK))
    o0, o1, oc = _elem_call(pp, pc_pad, order, m)

    triple_scores_sorted = skey.T.reshape(N_SORT)[:N_CAT]
    rel_class_sorted = oc[:N_CAT]
    rel_pair_sorted = jnp.stack([o0[:N_CAT], o1[:N_CAT]],
                                axis=1).astype(pair_dtype)
    class_prob_sorted = probs[:N_CAT, :50]
    return (triple_scores_sorted, rel_class_sorted, rel_pair_sorted,
            class_prob_sorted, obj_pred, obj_scores)
